# baseline (speedup 1.0000x reference)
import functools

import jax
import jax.numpy as jnp
from jax import lax
from jax.experimental import pallas as pl
from jax.experimental.pallas import tpu as pltpu
from jax.experimental.pallas import tpu_sc as plsc

VOCAB = 50257
SEQ = 2048
HID = 768
BATCH = 4

NUM_CORES = 2
NUM_SUBCORES = 16
NW = NUM_CORES * NUM_SUBCORES
S_PER_W = SEQ // NW
LANES = 16
VECS_PER_ROW = HID // LANES
G = 16
GROUPS = S_PER_W // G
NSET = 2


def _build():
    mesh = plsc.VectorSubcoreMesh(core_axis_name="c", subcore_axis_name="s")

    bufspec = pltpu.VMEM((G, HID), jnp.float32)

    @functools.partial(
        pl.kernel,
        mesh=mesh,
        out_type=jax.ShapeDtypeStruct((BATCH * SEQ, HID), jnp.float32),
        scratch_types=[
            pltpu.VMEM((BATCH, S_PER_W), jnp.int32),
            bufspec, bufspec, bufspec, bufspec, bufspec,
            bufspec, bufspec, bufspec, bufspec, bufspec,
            pltpu.SemaphoreType.DMA,
            pltpu.SemaphoreType.DMA,
            pltpu.SemaphoreType.DMA,
            pltpu.SemaphoreType.DMA,
            pltpu.SemaphoreType.DMA,
        ],
    )
    def embed(ids_hbm, table_hbm, pos_hbm, out_hbm,
              idx_v,
              p0, r00, r01, r02, r03,
              p1, r10, r11, r12, r13,
              isem, g0sem, g1sem, o0sem, o1sem):
        wid = lax.axis_index("s") * NUM_CORES + lax.axis_index("c")
        s_base = wid * S_PER_W
        posb = (p0, p1)
        rowb = ((r00, r01, r02, r03), (r10, r11, r12, r13))
        gsems = (g0sem, g1sem)
        osems = (o0sem, o1sem)

        def start_pos(g):
            st = g % NSET
            return pltpu.async_copy(
                pos_hbm.at[pl.ds(s_base + g * G, G)], posb[st], gsems[st])

        def start_gathers(g, pos_cp):
            st = g % NSET
            cps = [pos_cp]
            for b in range(BATCH):
                idx = idx_v.at[b, pl.ds(g * G, G)]
                cps.append(pltpu.async_copy(
                    table_hbm.at[idx], rowb[st][b], gsems[st]))
            return cps

        def start_group(g):
            return start_gathers(g, start_pos(g))

        pos0 = start_pos(0)
        pos1 = start_pos(1)
        id_copies = [
            pltpu.async_copy(
                ids_hbm.at[b, pl.ds(s_base, S_PER_W)], idx_v.at[b], isem)
            for b in range(BATCH)
        ]

        def start_outs(g):
            st = g % NSET
            return [
                pltpu.async_copy(
                    rowb[st][b],
                    out_hbm.at[pl.ds(b * SEQ + s_base + g * G, G)],
                    osems[st])
                for b in range(BATCH)
            ]

        gathers = [None] * GROUPS
        outs = [None] * GROUPS
        g0cps = [pos0]
        g1cps = [pos1]
        for b in range(BATCH):
            id_copies[b].wait()
            for g, cps in ((0, g0cps), (1, g1cps)):
                idx = idx_v.at[b, pl.ds(g * G, G)]
                cps.append(pltpu.async_copy(
                    table_hbm.at[idx], rowb[g % NSET][b], gsems[g % NSET]))
        gathers[0] = g0cps
        gathers[1] = g1cps

        for g in range(GROUPS):
            st = g % NSET
            for cp in gathers[g]:
                cp.wait()
            bufs = rowb[st]
            pbuf = posb[st]

            def _add(r, carry, _bufs=bufs, _pbuf=pbuf):
                for cc in range(VECS_PER_ROW):
                    sl = pl.ds(cc * LANES, LANES)
                    pv = _pbuf[r, sl]
                    for b in range(BATCH):
                        plsc.addupdate(_bufs[b].at[r, sl], pv)
                return carry

            lax.fori_loop(0, G, _add, 0)

            outs[g] = start_outs(g)
            if g + 2 < GROUPS:
                gn = g + 2
                stn = gn % NSET
                cps = [start_pos(gn)]
                for b in range(BATCH):
                    outs[g][b].wait()
                    idx = idx_v.at[b, pl.ds(gn * G, G)]
                    cps.append(pltpu.async_copy(
                        table_hbm.at[idx], rowb[stn][b], gsems[stn]))
                gathers[gn] = cps

        for g in (GROUPS - 2, GROUPS - 1):
            for cp in outs[g]:
                cp.wait()

    return embed


_embed = _build()


def kernel(input_ids, token_embeddings, position_embeddings):
    ids = input_ids.astype(jnp.int32)
    out = _embed(ids, token_embeddings, position_embeddings)
    return out.reshape(BATCH, SEQ, HID)


# speedup vs baseline: 1.0069x; 1.0069x over previous
import functools

import jax
import jax.numpy as jnp
from jax import lax
from jax.experimental import pallas as pl
from jax.experimental.pallas import tpu as pltpu
from jax.experimental.pallas import tpu_sc as plsc

VOCAB = 50257
SEQ = 2048
HID = 768
BATCH = 4

NUM_CORES = 2
NUM_SUBCORES = 16
NW = NUM_CORES * NUM_SUBCORES
S_PER_W = SEQ // NW
LANES = 16
VECS_PER_ROW = HID // LANES
G = 16
GROUPS = S_PER_W // G
NSET = 2


def _build():
    mesh = plsc.VectorSubcoreMesh(core_axis_name="c", subcore_axis_name="s")

    bufspec = pltpu.VMEM((G, HID), jnp.float32)

    @functools.partial(
        pl.kernel,
        mesh=mesh,
        out_type=jax.ShapeDtypeStruct((BATCH * SEQ, HID), jnp.float32),
        scratch_types=[
            pltpu.VMEM((BATCH, S_PER_W), jnp.int32),
            bufspec, bufspec, bufspec, bufspec, bufspec,
            bufspec, bufspec, bufspec, bufspec, bufspec,
            pltpu.SemaphoreType.DMA,
            pltpu.SemaphoreType.DMA,
            pltpu.SemaphoreType.DMA,
            pltpu.SemaphoreType.DMA,
            pltpu.SemaphoreType.DMA,
        ],
    )
    def embed(ids_hbm, table_hbm, pos_hbm, out_hbm,
              idx_v,
              p0, r00, r01, r02, r03,
              p1, r10, r11, r12, r13,
              isem, g0sem, g1sem, o0sem, o1sem):
        wid = lax.axis_index("s") * NUM_CORES + lax.axis_index("c")
        s_base = wid * S_PER_W
        posb = (p0, p1)
        rowb = ((r00, r01, r02, r03), (r10, r11, r12, r13))
        gsems = (g0sem, g1sem)
        osems = (o0sem, o1sem)

        def start_pos(g):
            st = g % NSET
            return pltpu.async_copy(
                pos_hbm.at[pl.ds(s_base + g * G, G)], posb[st], gsems[st])

        def start_gathers(g, pos_cp):
            st = g % NSET
            cps = [pos_cp]
            for b in range(BATCH):
                idx = idx_v.at[b, pl.ds(g * G, G)]
                cps.append(pltpu.async_copy(
                    table_hbm.at[idx], rowb[st][b], gsems[st]))
            return cps

        def start_group(g):
            return start_gathers(g, start_pos(g))

        pos0 = start_pos(0)
        pos1 = start_pos(1)
        id_copies = [
            pltpu.async_copy(
                ids_hbm.at[b, pl.ds(s_base, S_PER_W)], idx_v.at[b], isem)
            for b in range(BATCH)
        ]
        for cp in id_copies:
            cp.wait()

        def start_outs(g):
            st = g % NSET
            return [
                pltpu.async_copy(
                    rowb[st][b],
                    out_hbm.at[pl.ds(b * SEQ + s_base + g * G, G)],
                    osems[st])
                for b in range(BATCH)
            ]

        gathers = [None] * GROUPS
        outs = [None] * GROUPS
        gathers[0] = start_gathers(0, pos0)
        gathers[1] = start_gathers(1, pos1)

        for g in range(GROUPS):
            st = g % NSET
            for cp in gathers[g]:
                cp.wait()
            bufs = rowb[st]
            pbuf = posb[st]

            def _add(r, carry, _bufs=bufs, _pbuf=pbuf):
                for cc in range(VECS_PER_ROW):
                    sl = pl.ds(cc * LANES, LANES)
                    pv = _pbuf[r, sl]
                    for b in range(BATCH):
                        plsc.addupdate(_bufs[b].at[r, sl], pv)
                return carry

            lax.fori_loop(0, G, _add, 0)

            outs[g] = start_outs(g)
            if g + 2 < GROUPS:
                gn = g + 2
                stn = gn % NSET
                cps = [start_pos(gn)]
                for b in range(BATCH):
                    outs[g][b].wait()
                    idx = idx_v.at[b, pl.ds(gn * G, G)]
                    cps.append(pltpu.async_copy(
                        table_hbm.at[idx], rowb[stn][b], gsems[stn]))
                gathers[gn] = cps

        for g in (GROUPS - 2, GROUPS - 1):
            for cp in outs[g]:
                cp.wait()

    return embed


_embed = _build()


def kernel(input_ids, token_embeddings, position_embeddings):
    ids = input_ids.astype(jnp.int32)
    out = _embed(ids, token_embeddings, position_embeddings)
    return out.reshape(BATCH, SEQ, HID)
